# initial kernel scaffold (unmeasured)
import jax
import jax.numpy as jnp
from jax import lax
from jax.experimental import pallas as pl
from jax.experimental.pallas import tpu as pltpu

N_DEV = 4
BM = 1024
BN = 2048
BK = 1024


def _gemm_body(x_ref, w_ref, y_ref, amax_ref, amax_acc):
    j = pl.program_id(0)
    k = pl.program_id(1)
    nj = pl.num_programs(0)
    nk = pl.num_programs(1)

    @pl.when(jnp.logical_and(j == 0, k == 0))
    def _():
        amax_acc[0] = 0.0

    xb = x_ref[...].astype(jnp.bfloat16)
    wb = w_ref[...].astype(jnp.bfloat16)
    partial = jnp.dot(xb, wb, preferred_element_type=jnp.float32)

    @pl.when(k == 0)
    def _():
        y_ref[...] = partial

    @pl.when(k > 0)
    def _():
        y_ref[...] += partial

    @pl.when(k == nk - 1)
    def _():
        yb = jnp.maximum(y_ref[...], 0.0)
        y_ref[...] = yb
        amax_acc[0] = jnp.maximum(amax_acc[0], jnp.max(yb))

        @pl.when(j == nj - 1)
        def _():
            amax_ref[0, 0] = amax_acc[0]


def _gemm(x, w_mat):
    m, k_dim = x.shape
    _, n_dim = w_mat.shape
    return pl.pallas_call(
        _gemm_body,
        grid=(n_dim // BN, k_dim // BK),
        in_specs=[
            pl.BlockSpec((m, BK), lambda j, k: (0, k)),
            pl.BlockSpec((BK, BN), lambda j, k: (k, j)),
        ],
        out_specs=[
            pl.BlockSpec((m, BN), lambda j, k: (0, j)),
            pl.BlockSpec((1, 1), lambda j, k: (0, 0), memory_space=pltpu.SMEM),
        ],
        out_shape=[
            jax.ShapeDtypeStruct((m, n_dim), jnp.float32),
            jax.ShapeDtypeStruct((1, 1), jnp.float32),
        ],
        scratch_shapes=[pltpu.SMEM((1,), jnp.float32)],
        compiler_params=pltpu.CompilerParams(
            dimension_semantics=("arbitrary", "arbitrary"),
        ),
    )(x, w_mat)


def _comm_body(
    amax_ref,
    y_hbm,
    out_hbm,
    stage,
    q_send,
    q_recv,
    deq,
    amax_buf,
    amax_send_sems,
    amax_recv_sems,
    send_sems,
    recv_sems,
    local_sem,
):
    my = lax.axis_index("i")

    barrier = pltpu.get_barrier_semaphore()
    for d in range(1, N_DEV):
        peer = lax.rem(my + d, N_DEV)
        pl.semaphore_signal(
            barrier, inc=1, device_id=(peer,),
            device_id_type=pl.DeviceIdType.MESH,
        )
    pl.semaphore_wait(barrier, N_DEV - 1)

    amax_buf[N_DEV - 1] = jnp.full((8, 128), amax_ref[0, 0], jnp.float32)
    amax_rdmas = []
    for d in range(1, N_DEV):
        rdma = pltpu.make_async_remote_copy(
            src_ref=amax_buf.at[N_DEV - 1],
            dst_ref=amax_buf.at[N_DEV - 1 - d],
            send_sem=amax_send_sems.at[d - 1],
            recv_sem=amax_recv_sems.at[N_DEV - 1 - d],
            device_id=(lax.rem(my + d, N_DEV),),
            device_id_type=pl.DeviceIdType.MESH,
        )
        rdma.start()
        amax_rdmas.append(rdma)
    for d in range(1, N_DEV):
        recv = pltpu.make_async_remote_copy(
            src_ref=amax_buf.at[N_DEV - 1],
            dst_ref=amax_buf.at[d - 1],
            send_sem=amax_send_sems.at[d - 1],
            recv_sem=amax_recv_sems.at[d - 1],
            device_id=(lax.rem(my + d, N_DEV),),
            device_id_type=pl.DeviceIdType.MESH,
        )
        recv.wait_recv()
    for rdma in amax_rdmas:
        rdma.wait_send()

    g_amax = jnp.max(amax_buf[...])
    scale = g_amax / 448.0
    inv = 448.0 / g_amax

    data_rdmas = []
    for d in (1, 3, 2):
        dst = lax.rem(my + d, N_DEV)
        cp = pltpu.make_async_copy(
            y_hbm.at[:, pl.ds(dst * BN, BN)], stage, local_sem,
        )
        cp.start()
        cp.wait()
        q_send[d - 1] = (stage[...] * inv).astype(jnp.float8_e4m3fn)
        rdma = pltpu.make_async_remote_copy(
            src_ref=q_send.at[d - 1],
            dst_ref=q_recv.at[N_DEV - 1 - d],
            send_sem=send_sems.at[d - 1],
            recv_sem=recv_sems.at[N_DEV - 1 - d],
            device_id=(dst,),
            device_id_type=pl.DeviceIdType.MESH,
        )
        rdma.start()
        data_rdmas.append(rdma)

    cp = pltpu.make_async_copy(
        y_hbm.at[:, pl.ds(my * BN, BN)], stage, local_sem,
    )
    cp.start()
    cp.wait()
    q_own = (stage[...] * inv).astype(jnp.float8_e4m3fn)
    deq[...] = q_own.astype(jnp.float32) * scale
    cp = pltpu.make_async_copy(
        deq, out_hbm.at[pl.ds(my * BM, BM), :], local_sem,
    )
    cp.start()
    cp.wait()

    for d in (1, 3, 2):
        src = lax.rem(my + d, N_DEV)
        recv = pltpu.make_async_remote_copy(
            src_ref=q_send.at[d - 1],
            dst_ref=q_recv.at[d - 1],
            send_sem=send_sems.at[d - 1],
            recv_sem=recv_sems.at[d - 1],
            device_id=(src,),
            device_id_type=pl.DeviceIdType.MESH,
        )
        recv.wait_recv()
        deq[...] = q_recv[d - 1].astype(jnp.float32) * scale
        cp = pltpu.make_async_copy(
            deq, out_hbm.at[pl.ds(src * BM, BM), :], local_sem,
        )
        cp.start()
        cp.wait()

    for rdma in data_rdmas:
        rdma.wait_send()


def _comm(amax, y):
    m, n_dim = y.shape
    return pl.pallas_call(
        _comm_body,
        out_shape=jax.ShapeDtypeStruct((N_DEV * BM, BN), jnp.float32),
        in_specs=[
            pl.BlockSpec(memory_space=pltpu.SMEM),
            pl.BlockSpec(memory_space=pltpu.ANY),
        ],
        out_specs=pl.BlockSpec(memory_space=pltpu.ANY),
        scratch_shapes=[
            pltpu.VMEM((BM, BN), jnp.float32),
            pltpu.VMEM((3, BM, BN), jnp.float8_e4m3fn),
            pltpu.VMEM((3, BM, BN), jnp.float8_e4m3fn),
            pltpu.VMEM((BM, BN), jnp.float32),
            pltpu.VMEM((N_DEV, 8, 128), jnp.float32),
            pltpu.SemaphoreType.DMA((3,)),
            pltpu.SemaphoreType.DMA((3,)),
            pltpu.SemaphoreType.DMA((3,)),
            pltpu.SemaphoreType.DMA((3,)),
            pltpu.SemaphoreType.DMA,
        ],
        compiler_params=pltpu.CompilerParams(collective_id=0),
    )(amax, y)


def kernel(x, w_mat):
    y, amax = _gemm(x, w_mat)
    return _comm(amax, y)


# baseline (device time: 199845 ns/iter reference)
import jax
import jax.numpy as jnp
from jax import lax
from jax.experimental import pallas as pl
from jax.experimental.pallas import tpu as pltpu

N_DEV = 4
BM = 1024
BN = 2048
BK = 1024


def _gemm_body(x_ref, w_ref, y_ref, amax_ref, amax_acc):
    j = pl.program_id(0)
    k = pl.program_id(1)
    nj = pl.num_programs(0)
    nk = pl.num_programs(1)

    @pl.when(jnp.logical_and(j == 0, k == 0))
    def _():
        amax_acc[0] = 0.0

    xb = x_ref[...].astype(jnp.bfloat16)
    wb = w_ref[...].astype(jnp.bfloat16)
    partial = jnp.dot(xb, wb, preferred_element_type=jnp.float32)

    @pl.when(k == 0)
    def _():
        y_ref[...] = partial

    @pl.when(k > 0)
    def _():
        y_ref[...] += partial

    @pl.when(k == nk - 1)
    def _():
        yb = jnp.maximum(y_ref[...], 0.0)
        y_ref[...] = yb
        amax_acc[0] = jnp.maximum(amax_acc[0], jnp.max(yb))

        @pl.when(j == nj - 1)
        def _():
            amax_ref[0, 0] = amax_acc[0]


def _gemm(x, w_mat):
    m, k_dim = x.shape
    _, n_dim = w_mat.shape
    return pl.pallas_call(
        _gemm_body,
        grid=(n_dim // BN, k_dim // BK),
        in_specs=[
            pl.BlockSpec((m, BK), lambda j, k: (0, k)),
            pl.BlockSpec((BK, BN), lambda j, k: (k, j)),
        ],
        out_specs=[
            pl.BlockSpec((m, BN), lambda j, k: (0, j)),
            pl.BlockSpec((1, 1), lambda j, k: (0, 0), memory_space=pltpu.SMEM),
        ],
        out_shape=[
            jax.ShapeDtypeStruct((m, n_dim), jnp.float32),
            jax.ShapeDtypeStruct((1, 1), jnp.float32),
        ],
        scratch_shapes=[pltpu.SMEM((1,), jnp.float32)],
        compiler_params=pltpu.CompilerParams(
            dimension_semantics=("arbitrary", "arbitrary"),
            vmem_limit_bytes=60 * 1024 * 1024,
        ),
    )(x, w_mat)


def _comm_body(
    amax_ref,
    y_hbm,
    out_hbm,
    stage,
    q_send,
    q_recv,
    deq,
    amax_buf,
    amax_send_sems,
    amax_recv_sems,
    send_sems,
    recv_sems,
    local_sem,
):
    my = lax.axis_index("i")

    barrier = pltpu.get_barrier_semaphore()
    for d in range(1, N_DEV):
        peer = lax.rem(my + d, N_DEV)
        pl.semaphore_signal(
            barrier, inc=1, device_id=(peer,),
            device_id_type=pl.DeviceIdType.MESH,
        )
    pl.semaphore_wait(barrier, N_DEV - 1)

    amax_buf[N_DEV - 1] = jnp.full((8, 128), amax_ref[0, 0], jnp.float32)
    amax_rdmas = []
    for d in range(1, N_DEV):
        rdma = pltpu.make_async_remote_copy(
            src_ref=amax_buf.at[N_DEV - 1],
            dst_ref=amax_buf.at[N_DEV - 1 - d],
            send_sem=amax_send_sems.at[d - 1],
            recv_sem=amax_recv_sems.at[N_DEV - 1 - d],
            device_id=(lax.rem(my + d, N_DEV),),
            device_id_type=pl.DeviceIdType.MESH,
        )
        rdma.start()
        amax_rdmas.append(rdma)
    for d in range(1, N_DEV):
        recv = pltpu.make_async_remote_copy(
            src_ref=amax_buf.at[N_DEV - 1],
            dst_ref=amax_buf.at[d - 1],
            send_sem=amax_send_sems.at[d - 1],
            recv_sem=amax_recv_sems.at[d - 1],
            device_id=(lax.rem(my + d, N_DEV),),
            device_id_type=pl.DeviceIdType.MESH,
        )
        recv.wait_recv()
    for rdma in amax_rdmas:
        rdma.wait_send()

    g_amax = jnp.max(amax_buf[...])
    scale = g_amax / 448.0
    inv = 448.0 / g_amax

    data_rdmas = []
    for d in (1, 3, 2):
        dst = lax.rem(my + d, N_DEV)
        cp = pltpu.make_async_copy(
            y_hbm.at[:, pl.ds(dst * BN, BN)], stage, local_sem,
        )
        cp.start()
        cp.wait()
        q_send[d - 1] = (stage[...] * inv).astype(jnp.float8_e4m3fn)
        rdma = pltpu.make_async_remote_copy(
            src_ref=q_send.at[d - 1],
            dst_ref=q_recv.at[N_DEV - 1 - d],
            send_sem=send_sems.at[d - 1],
            recv_sem=recv_sems.at[N_DEV - 1 - d],
            device_id=(dst,),
            device_id_type=pl.DeviceIdType.MESH,
        )
        rdma.start()
        data_rdmas.append(rdma)

    cp = pltpu.make_async_copy(
        y_hbm.at[:, pl.ds(my * BN, BN)], stage, local_sem,
    )
    cp.start()
    cp.wait()
    q_own = (stage[...] * inv).astype(jnp.float8_e4m3fn)
    deq[...] = q_own.astype(jnp.float32) * scale
    cp = pltpu.make_async_copy(
        deq, out_hbm.at[pl.ds(my * BM, BM), :], local_sem,
    )
    cp.start()
    cp.wait()

    for d in (1, 3, 2):
        src = lax.rem(my + d, N_DEV)
        recv = pltpu.make_async_remote_copy(
            src_ref=q_send.at[d - 1],
            dst_ref=q_recv.at[d - 1],
            send_sem=send_sems.at[d - 1],
            recv_sem=recv_sems.at[d - 1],
            device_id=(src,),
            device_id_type=pl.DeviceIdType.MESH,
        )
        recv.wait_recv()
        deq[...] = q_recv[d - 1].astype(jnp.float32) * scale
        cp = pltpu.make_async_copy(
            deq, out_hbm.at[pl.ds(src * BM, BM), :], local_sem,
        )
        cp.start()
        cp.wait()

    for rdma in data_rdmas:
        rdma.wait_send()


def _comm(amax, y):
    m, n_dim = y.shape
    return pl.pallas_call(
        _comm_body,
        out_shape=jax.ShapeDtypeStruct((N_DEV * BM, BN), jnp.float32),
        in_specs=[
            pl.BlockSpec(memory_space=pltpu.SMEM),
            pl.BlockSpec(memory_space=pl.ANY),
        ],
        out_specs=pl.BlockSpec(memory_space=pl.ANY),
        scratch_shapes=[
            pltpu.VMEM((BM, BN), jnp.float32),
            pltpu.VMEM((3, BM, BN), jnp.float8_e4m3fn),
            pltpu.VMEM((3, BM, BN), jnp.float8_e4m3fn),
            pltpu.VMEM((BM, BN), jnp.float32),
            pltpu.VMEM((N_DEV, 8, 128), jnp.float32),
            pltpu.SemaphoreType.DMA((3,)),
            pltpu.SemaphoreType.DMA((3,)),
            pltpu.SemaphoreType.DMA((3,)),
            pltpu.SemaphoreType.DMA((3,)),
            pltpu.SemaphoreType.DMA,
        ],
        compiler_params=pltpu.CompilerParams(
            collective_id=0,
            vmem_limit_bytes=60 * 1024 * 1024,
        ),
    )(amax, y)


def kernel(x, w_mat):
    y, amax = _gemm(x, w_mat)
    return _comm(amax, y)


# device time: 173938 ns/iter; 1.1489x vs baseline; 1.1489x over previous
import jax
import jax.numpy as jnp
from jax import lax
from jax.experimental import pallas as pl
from jax.experimental.pallas import tpu as pltpu

N_DEV = 4
BM = 1024
BN = 2048
BK = 1024


GN = 512


def _gemm_body(x_ref, w_ref, y_ref, amax_ref, xbf, amax_acc):
    j = pl.program_id(0)
    nj = pl.num_programs(0)

    @pl.when(j == 0)
    def _():
        amax_acc[0] = 0.0
        xbf[...] = x_ref[...].astype(jnp.bfloat16)

    wb = w_ref[...].astype(jnp.bfloat16)
    yb = jnp.maximum(
        jnp.dot(xbf[...], wb, preferred_element_type=jnp.float32), 0.0
    )
    y_ref[...] = yb
    amax_acc[0] = jnp.maximum(amax_acc[0], jnp.max(yb))

    @pl.when(j == nj - 1)
    def _():
        amax_ref[0, 0] = amax_acc[0]


def _gemm(x, w_mat):
    m, k_dim = x.shape
    _, n_dim = w_mat.shape
    return pl.pallas_call(
        _gemm_body,
        grid=(n_dim // GN,),
        in_specs=[
            pl.BlockSpec((m, k_dim), lambda j: (0, 0)),
            pl.BlockSpec((k_dim, GN), lambda j: (0, j)),
        ],
        out_specs=[
            pl.BlockSpec((m, GN), lambda j: (0, j)),
            pl.BlockSpec((1, 1), lambda j: (0, 0), memory_space=pltpu.SMEM),
        ],
        out_shape=[
            jax.ShapeDtypeStruct((m, n_dim), jnp.float32),
            jax.ShapeDtypeStruct((1, 1), jnp.float32),
        ],
        scratch_shapes=[
            pltpu.VMEM((1024, 4096), jnp.bfloat16),
            pltpu.SMEM((1,), jnp.float32),
        ],
        compiler_params=pltpu.CompilerParams(
            dimension_semantics=("arbitrary",),
            vmem_limit_bytes=60 * 1024 * 1024,
        ),
    )(x, w_mat)


def _comm_body(
    amax_ref,
    y_hbm,
    out_hbm,
    stage,
    q_send,
    q_recv,
    deq,
    amax_buf,
    amax_send_sems,
    amax_recv_sems,
    send_sems,
    recv_sems,
    load_sems,
    store_sems,
):
    my = lax.axis_index("i")

    loads = {}
    for d in (1, 3, 2):
        dst = lax.rem(my + d, N_DEV)
        cp = pltpu.make_async_copy(
            y_hbm.at[:, pl.ds(dst * BN, BN)],
            stage.at[d - 1],
            load_sems.at[d - 1],
        )
        cp.start()
        loads[d] = cp
    own_load = pltpu.make_async_copy(
        y_hbm.at[:, pl.ds(my * BN, BN)], deq.at[0], load_sems.at[3],
    )
    own_load.start()

    barrier = pltpu.get_barrier_semaphore()
    for d in range(1, N_DEV):
        peer = lax.rem(my + d, N_DEV)
        pl.semaphore_signal(
            barrier, inc=1, device_id=(peer,),
            device_id_type=pl.DeviceIdType.MESH,
        )
    pl.semaphore_wait(barrier, N_DEV - 1)

    amax_buf[N_DEV - 1] = jnp.full((8, 128), amax_ref[0, 0], jnp.float32)
    amax_rdmas = []
    for d in range(1, N_DEV):
        rdma = pltpu.make_async_remote_copy(
            src_ref=amax_buf.at[N_DEV - 1],
            dst_ref=amax_buf.at[N_DEV - 1 - d],
            send_sem=amax_send_sems.at[d - 1],
            recv_sem=amax_recv_sems.at[N_DEV - 1 - d],
            device_id=(lax.rem(my + d, N_DEV),),
            device_id_type=pl.DeviceIdType.MESH,
        )
        rdma.start()
        amax_rdmas.append(rdma)
    for d in range(1, N_DEV):
        recv = pltpu.make_async_remote_copy(
            src_ref=amax_buf.at[N_DEV - 1],
            dst_ref=amax_buf.at[d - 1],
            send_sem=amax_send_sems.at[d - 1],
            recv_sem=amax_recv_sems.at[d - 1],
            device_id=(lax.rem(my + d, N_DEV),),
            device_id_type=pl.DeviceIdType.MESH,
        )
        recv.wait_recv()
    for rdma in amax_rdmas:
        rdma.wait_send()

    g_amax = jnp.max(amax_buf[...])
    scale = g_amax / 448.0
    inv = 448.0 / g_amax

    data_rdmas = []
    for d in (1, 3, 2):
        dst = lax.rem(my + d, N_DEV)
        loads[d].wait()
        q_send[d - 1] = (stage[d - 1] * inv).astype(jnp.float8_e4m3fn)
        rdma = pltpu.make_async_remote_copy(
            src_ref=q_send.at[d - 1],
            dst_ref=q_recv.at[N_DEV - 1 - d],
            send_sem=send_sems.at[d - 1],
            recv_sem=recv_sems.at[N_DEV - 1 - d],
            device_id=(dst,),
            device_id_type=pl.DeviceIdType.MESH,
        )
        rdma.start()
        data_rdmas.append(rdma)

    own_load.wait()
    deq[0] = ((deq[0] * inv).astype(jnp.float8_e4m3fn)).astype(
        jnp.float32
    ) * scale
    pending = [None, None]
    cp = pltpu.make_async_copy(
        deq.at[0], out_hbm.at[pl.ds(my * BM, BM), :], store_sems.at[0],
    )
    cp.start()
    pending[0] = cp

    for i, d in enumerate((1, 3, 2)):
        src = lax.rem(my + d, N_DEV)
        recv = pltpu.make_async_remote_copy(
            src_ref=q_send.at[d - 1],
            dst_ref=q_recv.at[d - 1],
            send_sem=send_sems.at[d - 1],
            recv_sem=recv_sems.at[d - 1],
            device_id=(src,),
            device_id_type=pl.DeviceIdType.MESH,
        )
        recv.wait_recv()
        slot = (i + 1) % 2
        if pending[slot] is not None:
            pending[slot].wait()
        deq[slot] = q_recv[d - 1].astype(jnp.float32) * scale
        cp = pltpu.make_async_copy(
            deq.at[slot], out_hbm.at[pl.ds(src * BM, BM), :],
            store_sems.at[slot],
        )
        cp.start()
        pending[slot] = cp

    for cp in pending:
        cp.wait()
    for rdma in data_rdmas:
        rdma.wait_send()


def _comm(amax, y):
    m, n_dim = y.shape
    return pl.pallas_call(
        _comm_body,
        out_shape=jax.ShapeDtypeStruct((N_DEV * BM, BN), jnp.float32),
        in_specs=[
            pl.BlockSpec(memory_space=pltpu.SMEM),
            pl.BlockSpec(memory_space=pl.ANY),
        ],
        out_specs=pl.BlockSpec(memory_space=pl.ANY),
        scratch_shapes=[
            pltpu.VMEM((3, BM, BN), jnp.float32),
            pltpu.VMEM((3, BM, BN), jnp.float8_e4m3fn),
            pltpu.VMEM((3, BM, BN), jnp.float8_e4m3fn),
            pltpu.VMEM((2, BM, BN), jnp.float32),
            pltpu.VMEM((N_DEV, 8, 128), jnp.float32),
            pltpu.SemaphoreType.DMA((3,)),
            pltpu.SemaphoreType.DMA((3,)),
            pltpu.SemaphoreType.DMA((3,)),
            pltpu.SemaphoreType.DMA((3,)),
            pltpu.SemaphoreType.DMA((4,)),
            pltpu.SemaphoreType.DMA((2,)),
        ],
        compiler_params=pltpu.CompilerParams(
            collective_id=0,
            vmem_limit_bytes=60 * 1024 * 1024,
        ),
    )(amax, y)


def kernel(x, w_mat):
    y, amax = _gemm(x, w_mat)
    return _comm(amax, y)
